# baseline (speedup 1.0000x reference)
import functools

import jax
import jax.numpy as jnp
from jax import lax
from jax.experimental import pallas as pl
from jax.experimental.pallas import tpu as pltpu
from jax.experimental.pallas import tpu_sc as plsc

_NC = 2
_NS = 16
_CH = 128


def _sc_aggregate(feature, edge_index, zrow):
    N, D = feature.shape
    E = edge_index.shape[1]
    NW = _NC * _NS
    rows_per_sub = ((N + _CH * _NS - 1) // (_CH * _NS)) * _CH
    NPAD = rows_per_sub * _NS

    G = -(-E // (_CH * NW))
    G = ((G + 3) // 4) * 4
    E_pad = G * NW * _CH
    if E_pad > E:
        pad = jnp.concatenate(
            [jnp.zeros((1, E_pad - E), jnp.int32),
             jnp.full((1, E_pad - E), N, jnp.int32)], axis=0)
        edge_index = jnp.concatenate([edge_index, pad], axis=1)
    edge_t = edge_index.reshape(2, G * NW, _CH).transpose(1, 0, 2)
    mesh = plsc.VectorSubcoreMesh(core_axis_name="c", subcore_axis_name="s")

    @functools.partial(
        pl.kernel,
        out_type=jax.ShapeDtypeStruct((_NC, NPAD, D), jnp.float32),
        mesh=mesh,
        scratch_types=[
            pltpu.VMEM((4, 2, _CH), jnp.int32),
            pltpu.VMEM((_CH, D), jnp.float32),
            pltpu.VMEM((_CH, D), jnp.float32),
            pltpu.VMEM_SHARED((NPAD, D), jnp.float32),
            [pltpu.SemaphoreType.DMA] * 4,
            [pltpu.SemaphoreType.DMA] * 2,
        ],
    )
    def agg_kernel(feat_hbm, edge_hbm, zrow_hbm, out_hbm,
                   idx, rows0, rows1, acc, isems, gsems):
        c = lax.axis_index("c")
        s = lax.axis_index("s")
        w = c * _NS + s
        c0 = w * G
        rows = (rows0, rows1)

        def idx_load(k, slot):
            pltpu.async_copy(edge_hbm.at[c0 + k], idx.at[slot], isems[slot])

        def idx_wait(k, slot):
            pltpu.make_async_copy(edge_hbm.at[c0 + k], idx.at[slot],
                                  isems[slot]).wait()

        def gather_start(slot, b):
            pltpu.async_copy(feat_hbm.at[idx.at[slot, 0]], rows[b], gsems[b])

        def gather_wait(slot, b):
            pltpu.make_async_copy(feat_hbm.at[idx.at[slot, 0]], rows[b],
                                  gsems[b]).wait()

        for k in range(3):
            idx_load(k, k)

        pltpu.sync_copy(zrow_hbm, rows0)

        @pl.loop(0, rows_per_sub // _CH)
        def _(j):
            pltpu.sync_copy(
                rows0, acc.at[pl.ds(s * rows_per_sub + j * _CH, _CH), :])

        plsc.subcore_barrier()

        @pl.loop(0, G, step=4)
        def _(j):
            for b in range(4):
                k = j + b
                idx_wait(k, b)
                gather_start(b, b % 2)
                pb = (b - 1) % 4

                @pl.when(k > 0)
                def _():
                    gather_wait(pb, pb % 2)
                    pltpu.sync_copy(rows[pb % 2], acc.at[idx.at[pb, 1]],
                                    add=True)

                @pl.when(k + 3 < G)
                def _():
                    idx_load(k + 3, (b + 3) % 4)

        lb = (G - 1) % 4
        gather_wait(lb, lb % 2)
        pltpu.sync_copy(rows[lb % 2], acc.at[idx.at[lb, 1]], add=True)

        plsc.subcore_barrier()

        pltpu.sync_copy(
            acc.at[pl.ds(s * rows_per_sub, rows_per_sub), :],
            out_hbm.at[c, pl.ds(s * rows_per_sub, rows_per_sub), :])

    return agg_kernel(feature, edge_t, zrow)


def _tc_fused(feature, partials, W1t, b1, W2t, b2, gamma, beta):
    N, D = feature.shape

    def body(f_ref, p_ref, w1_ref, b1_ref, w2_ref, b2_ref, g_ref, be_ref,
             o_ref):
        x = f_ref[...] + p_ref[0, pl.ds(0, N), :] + p_ref[1, pl.ds(0, N), :]
        h = jnp.dot(x, w1_ref[...], preferred_element_type=jnp.float32,
                    precision=lax.Precision.HIGHEST) + b1_ref[...]
        h = jnp.maximum(h, 0.0)
        h = jnp.dot(h, w2_ref[...], preferred_element_type=jnp.float32,
                    precision=lax.Precision.HIGHEST) + b2_ref[...]
        h = jnp.maximum(h, 0.0)
        mean = jnp.mean(h, axis=0, keepdims=True)
        var = jnp.mean(h * h, axis=0, keepdims=True) - mean * mean
        inv = lax.rsqrt(var + 1e-5)
        o_ref[...] = (h - mean) * inv * g_ref[...] + be_ref[...]

    return pl.pallas_call(
        body,
        out_shape=jax.ShapeDtypeStruct((N, D), jnp.float32),
    )(feature, partials, W1t, b1, W2t, b2, gamma, beta)


def kernel(feature, edge_index, W1, b1, W2, b2, gamma, beta):
    D = feature.shape[1]
    zrow = jnp.zeros((_CH, D), jnp.float32)
    partials = _sc_aggregate(feature, edge_index, zrow)
    return _tc_fused(feature, partials, W1.T, b1.reshape(1, D), W2.T,
                     b2.reshape(1, D), gamma.reshape(1, D),
                     beta.reshape(1, D))
